# Initial kernel scaffold; baseline (speedup 1.0000x reference)
import jax
import jax.numpy as jnp
from jax.experimental import pallas as pl


def kernel(x, W1, b1, W2, b2, Wg, bg):
    raise NotImplementedError("write your pallas kernel here")


# speedup vs baseline: 1.0413x; 1.0413x over previous
import functools

import jax
import jax.numpy as jnp
from jax import lax
from jax.experimental import pallas as pl
from jax.experimental.pallas import tpu as pltpu
from jax.experimental.pallas import tpu_sc as plsc

E = 8
TOPK = 2
B, D, H, O = 2048, 1024, 2048, 1024

T = 128
NPAD = B * TOPK + E * T
NT = NPAD // T

TB = 256

SC_CORES = 2
SC_SUBCORES = 16
NWORK = SC_CORES * SC_SUBCORES
def _sc_mesh():
    return plsc.VectorSubcoreMesh(
        core_axis_name="c", subcore_axis_name="s",
        num_cores=SC_CORES, num_subcores=SC_SUBCORES)

DISP_ROWS = NPAD // NWORK
DISP_CH = 32
COMB_TOK = B // NWORK
COMB_CH = 32


def _gate_body(x_ref, wg_ref, bg_ref, route_ref, usage_ref):
    t = pl.program_id(0)
    z = jnp.dot(x_ref[...], wg_ref[...], preferred_element_type=jnp.float32)
    z = z + bg_ref[...]
    m = jnp.max(z, axis=1, keepdims=True)
    p = jnp.exp(z - m)
    g = p / jnp.sum(p, axis=1, keepdims=True)

    @pl.when(t == 0)
    def _():
        usage_ref[...] = jnp.zeros_like(usage_ref)

    usage_ref[...] += jnp.sum(g, axis=0, keepdims=True)

    col = lax.broadcasted_iota(jnp.int32, g.shape, 1)
    v1 = jnp.max(g, axis=1, keepdims=True)
    i1 = jnp.min(jnp.where(g == v1, col, E), axis=1, keepdims=True)
    g2 = jnp.where(col == i1, -jnp.inf, g)
    v2 = jnp.max(g2, axis=1, keepdims=True)
    i2 = jnp.min(jnp.where(g2 == v2, col, E), axis=1, keepdims=True)
    s = v1 + v2
    w1 = v1 / s
    w2 = v2 / s
    route_ref[...] = jnp.where(
        col == 0, i1.astype(jnp.float32),
        jnp.where(col == 1, i2.astype(jnp.float32),
                  jnp.where(col == 2, w1, jnp.where(col == 3, w2, 0.0))))


def _gating(x, Wg, bg2):
    return pl.pallas_call(
        _gate_body,
        grid=(B // TB,),
        in_specs=[
            pl.BlockSpec((TB, D), lambda t: (t, 0)),
            pl.BlockSpec((D, E), lambda t: (0, 0)),
            pl.BlockSpec((1, E), lambda t: (0, 0)),
        ],
        out_specs=[
            pl.BlockSpec((TB, E), lambda t: (t, 0)),
            pl.BlockSpec((1, E), lambda t: (0, 0)),
        ],
        out_shape=[
            jax.ShapeDtypeStruct((B, E), jnp.float32),
            jax.ShapeDtypeStruct((1, E), jnp.float32),
        ],
    )(x, Wg, bg2)


@functools.cache
def _build_dispatch():
    @functools.partial(
        pl.kernel,
        mesh=_sc_mesh(),
        out_type=jax.ShapeDtypeStruct((NPAD, D), jnp.float32),
        scratch_types=[
            pltpu.VMEM((DISP_CH,), jnp.int32),
            pltpu.VMEM((DISP_CH, D), jnp.float32),
            pltpu.SemaphoreType.DMA,
        ],
    )
    def _dispatch(x_hbm, src_hbm, xp_hbm, idx_v, rows_v, sem):
        wid = lax.axis_index("s") * SC_CORES + lax.axis_index("c")
        base = wid * DISP_ROWS

        def chunk(i, carry):
            off = base + i * DISP_CH
            pltpu.sync_copy(src_hbm.at[pl.ds(off, DISP_CH)], idx_v)
            pltpu.async_copy(x_hbm.at[idx_v], rows_v, sem).wait()
            pltpu.sync_copy(rows_v, xp_hbm.at[pl.ds(off, DISP_CH)])
            return carry

        lax.fori_loop(0, DISP_ROWS // DISP_CH, chunk, 0)

    return _dispatch


def _ffn_body(te_ref, vl_ref, fi_ref, xp_ref, w1_ref, b1_ref, w2_ref, b2_ref,
              wp_ref, out_ref, sums_ref):
    t = pl.program_id(0)
    h = jnp.dot(xp_ref[...], w1_ref[0], preferred_element_type=jnp.float32)
    h = jnp.maximum(h + b1_ref[0], 0.0)
    eo = jnp.dot(h, w2_ref[0], preferred_element_type=jnp.float32)
    eo = eo + b2_ref[0]

    rows = lax.broadcasted_iota(jnp.int32, (T, 1), 0)
    mask = rows < vl_ref[t]
    contrib = jnp.sum(jnp.where(mask, eo, 0.0), axis=0, keepdims=True)

    @pl.when(fi_ref[t] == 1)
    def _():
        sums_ref[0] = contrib

    @pl.when(fi_ref[t] == 0)
    def _():
        sums_ref[0] += contrib

    out_ref[...] = eo * wp_ref[...]


def _ffn(te, vl, fi, xpad, W1, b1, W2, b2, wpad):
    grid_spec = pltpu.PrefetchScalarGridSpec(
        num_scalar_prefetch=3,
        grid=(NT,),
        in_specs=[
            pl.BlockSpec((T, D), lambda t, te, vl, fi: (t, 0)),
            pl.BlockSpec((1, D, H), lambda t, te, vl, fi: (te[t], 0, 0)),
            pl.BlockSpec((1, 1, H), lambda t, te, vl, fi: (te[t], 0, 0)),
            pl.BlockSpec((1, H, O), lambda t, te, vl, fi: (te[t], 0, 0)),
            pl.BlockSpec((1, 1, O), lambda t, te, vl, fi: (te[t], 0, 0)),
            pl.BlockSpec((T, 1), lambda t, te, vl, fi: (t, 0)),
        ],
        out_specs=[
            pl.BlockSpec((T, O), lambda t, te, vl, fi: (t, 0)),
            pl.BlockSpec((1, 1, O), lambda t, te, vl, fi: (te[t], 0, 0)),
        ],
    )
    out_pad, sums3 = pl.pallas_call(
        _ffn_body,
        grid_spec=grid_spec,
        out_shape=[
            jax.ShapeDtypeStruct((NPAD, O), jnp.float32),
            jax.ShapeDtypeStruct((E, 1, O), jnp.float32),
        ],
    )(te, vl, fi, xpad, W1, b1.reshape(E, 1, H), W2, b2.reshape(E, 1, O),
      wpad)
    return out_pad, sums3.reshape(E, O)


@functools.cache
def _build_combine():
    @functools.partial(
        pl.kernel,
        mesh=_sc_mesh(),
        out_type=jax.ShapeDtypeStruct((B, O), jnp.float32),
        scratch_types=[
            pltpu.VMEM((COMB_CH,), jnp.int32),
            pltpu.VMEM((COMB_CH,), jnp.int32),
            pltpu.VMEM((COMB_CH, O), jnp.float32),
            pltpu.VMEM((COMB_CH, O), jnp.float32),
            pltpu.SemaphoreType.DMA,
            pltpu.SemaphoreType.DMA,
        ],
    )
    def _combine(op_hbm, p0_hbm, p1_hbm, out_hbm, i0_v, i1_v, g0_v, g1_v,
                 s0, s1):
        wid = lax.axis_index("s") * SC_CORES + lax.axis_index("c")
        base = wid * COMB_TOK

        def chunk(i, carry):
            off = base + i * COMB_CH
            pltpu.sync_copy(p0_hbm.at[pl.ds(off, COMB_CH)], i0_v)
            pltpu.sync_copy(p1_hbm.at[pl.ds(off, COMB_CH)], i1_v)
            c0 = pltpu.async_copy(op_hbm.at[i0_v], g0_v, s0)
            c1 = pltpu.async_copy(op_hbm.at[i1_v], g1_v, s1)
            c0.wait()
            c1.wait()

            def addrow(r, c2):
                def addcol(c, c3):
                    g0_v[r, pl.ds(c * 16, 16)] += g1_v[r, pl.ds(c * 16, 16)]
                    return c3
                return lax.fori_loop(0, O // 16, addcol, c2)

            lax.fori_loop(0, COMB_CH, addrow, 0)
            pltpu.sync_copy(g0_v, out_hbm.at[pl.ds(off, COMB_CH)])
            return carry

        lax.fori_loop(0, COMB_TOK // COMB_CH, chunk, 0)

    return _combine


def _loss_body(sums_ref, ccol_ref, crow_ref, usage_ref, dl_ref, eq_ref):
    ccol = ccol_ref[...]
    avg = jnp.where(ccol > 0.0, sums_ref[...] / jnp.maximum(ccol, 1.0), 0.0)
    coli = lax.broadcasted_iota(jnp.int32, (E, E), 1)
    rowi = lax.broadcasted_iota(jnp.int32, (E, E), 0)
    ssq = jnp.zeros((E, E), jnp.float32)
    for e in range(E):
        diff = avg - avg[e:e + 1, :]
        ssq = jnp.where(coli == e,
                        jnp.sum(diff * diff, axis=1, keepdims=True), ssq)
    sim = jnp.exp(-ssq / 2.0)
    pair = (rowi < coli) & (ccol > 0.0) & (crow_ref[...] > 0.0)
    dl_ref[...] = jnp.sum(jnp.where(pair, sim, 0.0), keepdims=True)

    u = usage_ref[...] / B - 1.0 / E
    eq_ref[...] = jnp.sqrt(jnp.sum(u * u, keepdims=True))


def _losses(sums, ccol, crow, usage):
    return pl.pallas_call(
        _loss_body,
        out_shape=[
            jax.ShapeDtypeStruct((1, 1), jnp.float32),
            jax.ShapeDtypeStruct((1, 1), jnp.float32),
        ],
    )(sums, ccol, crow, usage)


def kernel(x, W1, b1, W2, b2, Wg, bg):
    route, usage = _gating(x, Wg, bg.reshape(1, E))
    topi = route[:, :2].astype(jnp.int32)
    w = route[:, 2:4]

    eids = topi.reshape(-1)
    order = jnp.argsort(eids, stable=True)
    sorted_e = eids[order]
    counts = jnp.zeros((E,), jnp.int32).at[eids].add(1)
    tiles_per = (counts + (T - 1)) // T
    tile_start = jnp.concatenate(
        [jnp.zeros((1,), jnp.int32), jnp.cumsum(tiles_per)[:-1]])
    padded_start = tile_start * T
    seg_start = jnp.concatenate(
        [jnp.zeros((1,), jnp.int32), jnp.cumsum(counts)[:-1]])
    rank = jnp.arange(B * TOPK, dtype=jnp.int32) - seg_start[sorted_e]
    dst = padded_start[sorted_e] + rank
    tok = (order // TOPK).astype(jnp.int32)
    src = jnp.zeros((NPAD,), jnp.int32).at[dst].set(tok)
    wpad = jnp.zeros((NPAD,), jnp.float32).at[dst].set(
        w.reshape(-1)[order]).reshape(NPAD, 1)
    pos = jnp.zeros((B * TOPK,), jnp.int32).at[order].set(dst)
    p0 = pos[0::2]
    p1 = pos[1::2]

    tids = jnp.arange(NT, dtype=jnp.int32)
    te = jnp.minimum(
        jnp.searchsorted(jnp.cumsum(tiles_per), tids, side="right"),
        E - 1).astype(jnp.int32)
    vl = jnp.clip(counts[te] - (tids - tile_start[te]) * T, 0, T
                  ).astype(jnp.int32)
    fi = jnp.concatenate(
        [jnp.ones((1,), jnp.int32), (te[1:] != te[:-1]).astype(jnp.int32)])

    xpad = _build_dispatch()(x, src)
    out_pad, sums = _ffn(te, vl, fi, xpad, W1, b1, W2, b2, wpad)
    final = _build_combine()(out_pad, p0, p1)

    cf = counts.astype(jnp.float32)
    dl, eq = _losses(sums, cf.reshape(E, 1), cf.reshape(1, E), usage)
    return (final, dl[0, 0], eq[0, 0])


# speedup vs baseline: 1.1104x; 1.0663x over previous
import functools

import jax
import jax.numpy as jnp
from jax import lax
from jax.experimental import pallas as pl
from jax.experimental.pallas import tpu as pltpu
from jax.experimental.pallas import tpu_sc as plsc

E = 8
TOPK = 2
B, D, H, O = 2048, 1024, 2048, 1024

T = 128
NPAD = B * TOPK + E * T
NT = NPAD // T

TB = 256

SC_CORES = 2
SC_SUBCORES = 16
NWORK = SC_CORES * SC_SUBCORES
def _sc_mesh():
    return plsc.VectorSubcoreMesh(
        core_axis_name="c", subcore_axis_name="s",
        num_cores=SC_CORES, num_subcores=SC_SUBCORES)

DISP_ROWS = NPAD // NWORK
DISP_CH = 40
DISP_NCH = DISP_ROWS // DISP_CH
COMB_TOK = B // NWORK
COMB_CH = 16
COMB_NCH = COMB_TOK // COMB_CH


def _gate_body(x_ref, wg_ref, bg_ref, route_ref, usage_ref):
    t = pl.program_id(0)
    z = jnp.dot(x_ref[...], wg_ref[...], preferred_element_type=jnp.float32)
    z = z + bg_ref[...]
    m = jnp.max(z, axis=1, keepdims=True)
    p = jnp.exp(z - m)
    g = p / jnp.sum(p, axis=1, keepdims=True)

    @pl.when(t == 0)
    def _():
        usage_ref[...] = jnp.zeros_like(usage_ref)

    usage_ref[...] += jnp.sum(g, axis=0, keepdims=True)

    col = lax.broadcasted_iota(jnp.int32, g.shape, 1)
    v1 = jnp.max(g, axis=1, keepdims=True)
    i1 = jnp.min(jnp.where(g == v1, col, E), axis=1, keepdims=True)
    g2 = jnp.where(col == i1, -jnp.inf, g)
    v2 = jnp.max(g2, axis=1, keepdims=True)
    i2 = jnp.min(jnp.where(g2 == v2, col, E), axis=1, keepdims=True)
    s = v1 + v2
    w1 = v1 / s
    w2 = v2 / s
    route_ref[...] = jnp.where(
        col == 0, i1.astype(jnp.float32),
        jnp.where(col == 1, i2.astype(jnp.float32),
                  jnp.where(col == 2, w1, jnp.where(col == 3, w2, 0.0))))


def _gating(x, Wg, bg2):
    return pl.pallas_call(
        _gate_body,
        grid=(B // TB,),
        in_specs=[
            pl.BlockSpec((TB, D), lambda t: (t, 0)),
            pl.BlockSpec((D, E), lambda t: (0, 0)),
            pl.BlockSpec((1, E), lambda t: (0, 0)),
        ],
        out_specs=[
            pl.BlockSpec((TB, E), lambda t: (t, 0)),
            pl.BlockSpec((1, E), lambda t: (0, 0)),
        ],
        out_shape=[
            jax.ShapeDtypeStruct((B, E), jnp.float32),
            jax.ShapeDtypeStruct((1, E), jnp.float32),
        ],
    )(x, Wg, bg2)


@functools.cache
def _build_dispatch():
    @functools.partial(
        pl.kernel,
        mesh=_sc_mesh(),
        out_type=jax.ShapeDtypeStruct((NPAD, D), jnp.float32),
        scratch_types=[
            pltpu.VMEM((DISP_ROWS,), jnp.int32),
            pltpu.VMEM((DISP_CH, D), jnp.float32),
            pltpu.VMEM((DISP_CH, D), jnp.float32),
            pltpu.SemaphoreType.DMA,
            pltpu.SemaphoreType.DMA,
            pltpu.SemaphoreType.DMA,
            pltpu.SemaphoreType.DMA,
        ],
    )
    def _dispatch(x_hbm, src_hbm, xp_hbm, idx_v, r0, r1, sg0, sg1, ss0, ss1):
        wid = lax.axis_index("s") * SC_CORES + lax.axis_index("c")
        base = wid * DISP_ROWS
        rows = (r0, r1)
        gsems = (sg0, sg1)
        ssems = (ss0, ss1)

        pltpu.sync_copy(src_hbm.at[pl.ds(base, DISP_ROWS)], idx_v)

        def gather(c):
            return pltpu.async_copy(
                x_hbm.at[idx_v.at[pl.ds(c * DISP_CH, DISP_CH)]],
                rows[c % 2], gsems[c % 2])

        g = gather(0)
        scat = [None] * DISP_NCH
        for c in range(DISP_NCH):
            g.wait()
            scat[c] = pltpu.async_copy(
                rows[c % 2], xp_hbm.at[pl.ds(base + c * DISP_CH, DISP_CH)],
                ssems[c % 2])
            if c + 1 < DISP_NCH:
                if c >= 1:
                    scat[c - 1].wait()
                g = gather(c + 1)
        scat[DISP_NCH - 2].wait()
        scat[DISP_NCH - 1].wait()

    return _dispatch


def _ffn_body(te_ref, vl_ref, fi_ref, xp_ref, w1_ref, b1_ref, w2_ref, b2_ref,
              wp_ref, out_ref, sums_ref):
    t = pl.program_id(0)

    @pl.when(vl_ref[t] > 0)
    def _():
        h = jnp.dot(xp_ref[...], w1_ref[0],
                    preferred_element_type=jnp.float32)
        h = jnp.maximum(h + b1_ref[0], 0.0)
        eo = jnp.dot(h, w2_ref[0], preferred_element_type=jnp.float32)
        eo = eo + b2_ref[0]

        rows = lax.broadcasted_iota(jnp.int32, (T, 1), 0)
        mask = rows < vl_ref[t]
        contrib = jnp.sum(jnp.where(mask, eo, 0.0), axis=0, keepdims=True)

        @pl.when(fi_ref[t] == 1)
        def _():
            sums_ref[0] = contrib

        @pl.when(fi_ref[t] == 0)
        def _():
            sums_ref[0] += contrib

        out_ref[...] = eo * wp_ref[...]

    @pl.when(vl_ref[t] == 0)
    def _():
        out_ref[...] = jnp.zeros_like(out_ref)

        @pl.when(fi_ref[t] == 1)
        def _():
            sums_ref[0] = jnp.zeros_like(sums_ref[0])


def _ffn(te, vl, fi, xpad, W1, b1, W2, b2, wpad):
    grid_spec = pltpu.PrefetchScalarGridSpec(
        num_scalar_prefetch=3,
        grid=(NT,),
        in_specs=[
            pl.BlockSpec((T, D), lambda t, te, vl, fi: (t, 0)),
            pl.BlockSpec((1, D, H), lambda t, te, vl, fi: (te[t], 0, 0)),
            pl.BlockSpec((1, 1, H), lambda t, te, vl, fi: (te[t], 0, 0)),
            pl.BlockSpec((1, H, O), lambda t, te, vl, fi: (te[t], 0, 0)),
            pl.BlockSpec((1, 1, O), lambda t, te, vl, fi: (te[t], 0, 0)),
            pl.BlockSpec((T, 1), lambda t, te, vl, fi: (t, 0)),
        ],
        out_specs=[
            pl.BlockSpec((T, O), lambda t, te, vl, fi: (t, 0)),
            pl.BlockSpec((1, 1, O), lambda t, te, vl, fi: (te[t], 0, 0)),
        ],
    )
    out_pad, sums3 = pl.pallas_call(
        _ffn_body,
        grid_spec=grid_spec,
        out_shape=[
            jax.ShapeDtypeStruct((NPAD, O), jnp.float32),
            jax.ShapeDtypeStruct((E, 1, O), jnp.float32),
        ],
    )(te, vl, fi, xpad, W1, b1.reshape(E, 1, H), W2, b2.reshape(E, 1, O),
      wpad)
    return out_pad, sums3.reshape(E, O)


@functools.cache
def _build_combine():
    @functools.partial(
        pl.kernel,
        mesh=_sc_mesh(),
        out_type=jax.ShapeDtypeStruct((B, O), jnp.float32),
        scratch_types=[
            pltpu.VMEM((COMB_TOK,), jnp.int32),
            pltpu.VMEM((COMB_TOK,), jnp.int32),
            pltpu.VMEM((COMB_CH, O), jnp.float32),
            pltpu.VMEM((COMB_CH, O), jnp.float32),
            pltpu.VMEM((COMB_CH, O), jnp.float32),
            pltpu.VMEM((COMB_CH, O), jnp.float32),
            pltpu.SemaphoreType.DMA,
            pltpu.SemaphoreType.DMA,
            pltpu.SemaphoreType.DMA,
            pltpu.SemaphoreType.DMA,
            pltpu.SemaphoreType.DMA,
            pltpu.SemaphoreType.DMA,
        ],
    )
    def _combine(op_hbm, p0_hbm, p1_hbm, out_hbm, i0_v, i1_v,
                 a0, b0, a1, b1, sa0, sb0, sa1, sb1, so0, so1):
        wid = lax.axis_index("s") * SC_CORES + lax.axis_index("c")
        base = wid * COMB_TOK
        abuf = (a0, a1)
        bbuf = (b0, b1)
        asem = (sa0, sa1)
        bsem = (sb0, sb1)
        osem = (so0, so1)

        pltpu.sync_copy(p0_hbm.at[pl.ds(base, COMB_TOK)], i0_v)
        pltpu.sync_copy(p1_hbm.at[pl.ds(base, COMB_TOK)], i1_v)

        def gathers(c):
            s = c % 2
            i0 = i0_v[pl.ds(c * COMB_CH, COMB_CH)]
            i1 = i1_v[pl.ds(c * COMB_CH, COMB_CH)]
            return (pltpu.async_copy(op_hbm.at[i0], abuf[s], asem[s]),
                    pltpu.async_copy(op_hbm.at[i1], bbuf[s], bsem[s]))

        g = gathers(0)
        scat = [None] * COMB_NCH
        for c in range(COMB_NCH):
            s = c % 2
            g[0].wait()
            g[1].wait()
            if c + 1 < COMB_NCH:
                if c >= 1:
                    scat[c - 1].wait()
                g = gathers(c + 1)

            def addrow(r, carry, s=s):
                for col in range(O // 16):
                    sl = pl.ds(col * 16, 16)
                    abuf[s][r, sl] += bbuf[s][r, sl]
                return carry

            lax.fori_loop(0, COMB_CH, addrow, 0)
            scat[c] = pltpu.async_copy(
                abuf[s], out_hbm.at[pl.ds(base + c * COMB_CH, COMB_CH)],
                osem[s])
        scat[COMB_NCH - 2].wait()
        scat[COMB_NCH - 1].wait()

    return _combine


def _loss_body(sums_ref, ccol_ref, crow_ref, usage_ref, dl_ref, eq_ref):
    ccol = ccol_ref[...]
    avg = jnp.where(ccol > 0.0, sums_ref[...] / jnp.maximum(ccol, 1.0), 0.0)
    coli = lax.broadcasted_iota(jnp.int32, (E, E), 1)
    rowi = lax.broadcasted_iota(jnp.int32, (E, E), 0)
    ssq = jnp.zeros((E, E), jnp.float32)
    for e in range(E):
        diff = avg - avg[e:e + 1, :]
        ssq = jnp.where(coli == e,
                        jnp.sum(diff * diff, axis=1, keepdims=True), ssq)
    sim = jnp.exp(-ssq / 2.0)
    pair = (rowi < coli) & (ccol > 0.0) & (crow_ref[...] > 0.0)
    dl_ref[...] = jnp.sum(jnp.where(pair, sim, 0.0), keepdims=True)

    u = usage_ref[...] / B - 1.0 / E
    eq_ref[...] = jnp.sqrt(jnp.sum(u * u, keepdims=True))


def _losses(sums, ccol, crow, usage):
    return pl.pallas_call(
        _loss_body,
        out_shape=[
            jax.ShapeDtypeStruct((1, 1), jnp.float32),
            jax.ShapeDtypeStruct((1, 1), jnp.float32),
        ],
    )(sums, ccol, crow, usage)


def kernel(x, W1, b1, W2, b2, Wg, bg):
    route, usage = _gating(x, Wg, bg.reshape(1, E))
    topi = route[:, :2].astype(jnp.int32)
    w = route[:, 2:4]

    eids = topi.reshape(-1)
    order = jnp.argsort(eids, stable=True)
    sorted_e = eids[order]
    counts = jnp.zeros((E,), jnp.int32).at[eids].add(1)
    tiles_per = (counts + (T - 1)) // T
    tile_start = jnp.concatenate(
        [jnp.zeros((1,), jnp.int32), jnp.cumsum(tiles_per)[:-1]])
    padded_start = tile_start * T
    seg_start = jnp.concatenate(
        [jnp.zeros((1,), jnp.int32), jnp.cumsum(counts)[:-1]])
    rank = jnp.arange(B * TOPK, dtype=jnp.int32) - seg_start[sorted_e]
    dst = padded_start[sorted_e] + rank
    tok = (order // TOPK).astype(jnp.int32)
    src = jnp.zeros((NPAD,), jnp.int32).at[dst].set(tok)
    wpad = jnp.zeros((NPAD,), jnp.float32).at[dst].set(
        w.reshape(-1)[order]).reshape(NPAD, 1)
    pos = jnp.zeros((B * TOPK,), jnp.int32).at[order].set(dst)
    p0 = pos[0::2]
    p1 = pos[1::2]

    tids = jnp.arange(NT, dtype=jnp.int32)
    te = jnp.minimum(
        jnp.searchsorted(jnp.cumsum(tiles_per), tids, side="right"),
        E - 1).astype(jnp.int32)
    vl = jnp.clip(counts[te] - (tids - tile_start[te]) * T, 0, T
                  ).astype(jnp.int32)
    fi = jnp.concatenate(
        [jnp.ones((1,), jnp.int32), (te[1:] != te[:-1]).astype(jnp.int32)])

    xpad = _build_dispatch()(x, src)
    out_pad, sums = _ffn(te, vl, fi, xpad, W1, b1, W2, b2, wpad)
    final = _build_combine()(out_pad, p0, p1)

    cf = counts.astype(jnp.float32)
    dl, eq = _losses(sums, cf.reshape(E, 1), cf.reshape(1, E), usage)
    return (final, dl[0, 0], eq[0, 0])
